# baseline (speedup 1.0000x reference)
import functools

import jax
import jax.numpy as jnp
from jax import lax
from jax.experimental import pallas as pl
from jax.experimental.pallas import tpu as pltpu
from jax.experimental.pallas import tpu_sc as plsc

N = 10000
E = 320000
D = 128
DE = 16
L = 7

NC = 2
NS = 16
NW = NC * NS
EH = E // 2
CH = 40
EH_PAD = NW * 128 * CH
N_PAD = 10240
ROWS_PER_TILE = N_PAD // NS

_MESH = plsc.VectorSubcoreMesh(
    core_axis_name="c", subcore_axis_name="s", num_cores=NC, num_subcores=NS)


def _pipeline(nch, start, process):
    start(0, 0)
    start(1, 1)
    process(0, 0, False, True)
    process(1, 1, False, True)
    p_max = (nch - 4) // 2

    def body(p, carry):
        j = 2 * p
        process(j, 0, True, True)
        process(j + 1, 1, True, True)
        return carry

    lax.fori_loop(1, p_max + 1, body, 0)
    for j in range(2 * (p_max + 1), nch):
        process(j, j % 2, True, j + 2 < nch)


def _make_gather(e_cnt):
    per_w = e_cnt // NW
    nch = per_w // CH

    @functools.partial(
        pl.kernel,
        out_type=jax.ShapeDtypeStruct((e_cnt, D), jnp.float32),
        mesh=_MESH,
        scratch_types=[
            pltpu.VMEM((nch, CH), jnp.int32),
            pltpu.VMEM((nch, CH), jnp.int32),
            pltpu.VMEM((CH, D), jnp.float32),
            pltpu.VMEM((CH, D), jnp.float32),
            pltpu.VMEM((CH, D), jnp.float32),
            pltpu.VMEM((CH, D), jnp.float32),
            pltpu.VMEM((CH, D), jnp.float32),
            pltpu.VMEM((CH, D), jnp.float32),
            pltpu.SemaphoreType.DMA,
            pltpu.SemaphoreType.DMA,
            pltpu.SemaphoreType.DMA,
            pltpu.SemaphoreType.DMA,
            pltpu.SemaphoreType.DMA,
            pltpu.SemaphoreType.DMA,
        ],
    )
    def gather(x_hbm, src_hbm, dst_hbm, out_hbm,
               sidx, didx, a0, a1, b0, b1, o0, o1,
               sa0, sa1, sb0, sb1, so0, so1):
        c = lax.axis_index("c")
        s = lax.axis_index("s")
        wid = s * NC + c
        base = wid * per_w
        pltpu.sync_copy(src_hbm.at[wid, :, :], sidx)
        pltpu.sync_copy(dst_hbm.at[wid, :, :], didx)

        ab = ((a0, b0, o0, sa0, sb0, so0), (a1, b1, o1, sa1, sb1, so1))

        def start(j, ph):
            a, b, _, sa, sb, _ = ab[ph]
            pltpu.async_copy(x_hbm.at[sidx.at[j]], a, sa)
            pltpu.async_copy(x_hbm.at[didx.at[j]], b, sb)

        def process(j, ph, wait_wb, start_next):
            a, b, o, sa, sb, so = ab[ph]
            pltpu.make_async_copy(x_hbm.at[sidx.at[j]], a, sa).wait()
            pltpu.make_async_copy(x_hbm.at[didx.at[j]], b, sb).wait()
            if wait_wb:
                pltpu.make_async_copy(o, out_hbm.at[pl.ds(base, CH)],
                                      so).wait()

            def row(i, carry):
                for k in range(D // 16):
                    sl = pl.ds(k * 16, 16)
                    o[i, sl] = a[i, sl] - b[i, sl]
                return carry

            lax.fori_loop(0, CH, row, 0, unroll=4)
            if start_next:
                start(j + 2, ph)
            pltpu.async_copy(o, out_hbm.at[pl.ds(base + j * CH, CH)], so)

        _pipeline(nch, start, process)
        pltpu.make_async_copy(o0, out_hbm.at[pl.ds(base, CH)], so0).wait()
        pltpu.make_async_copy(o1, out_hbm.at[pl.ds(base, CH)], so1).wait()

    return gather


def _make_scatter(e_cnt):
    per_w = e_cnt // NW
    nch = per_w // CH

    @functools.partial(
        pl.kernel,
        out_type=jax.ShapeDtypeStruct((NC, N_PAD, D), jnp.float32),
        mesh=_MESH,
        scratch_types=[
            pltpu.VMEM((nch, CH), jnp.int32),
            pltpu.VMEM((CH, D), jnp.float32),
            pltpu.VMEM((CH, D), jnp.float32),
            pltpu.VMEM_SHARED((N_PAD, D), jnp.float32),
            pltpu.SemaphoreType.DMA,
            pltpu.SemaphoreType.DMA,
        ],
    )
    def scatter(msg_hbm, dst_hbm, zeros_hbm, out_hbm,
                idx_v, m0, m1, acc, sm0, sm1):
        c = lax.axis_index("c")
        s = lax.axis_index("s")
        wid = c * NS + s
        row0 = s * ROWS_PER_TILE
        base = wid * per_w
        pltpu.sync_copy(dst_hbm.at[wid, :, :], idx_v)
        pltpu.sync_copy(zeros_hbm.at[pl.ds(row0, ROWS_PER_TILE)],
                        acc.at[pl.ds(row0, ROWS_PER_TILE)])
        plsc.subcore_barrier()

        ms = ((m0, sm0), (m1, sm1))

        def start(j, ph):
            m, sm = ms[ph]
            pltpu.async_copy(msg_hbm.at[pl.ds(base + j * CH, CH)], m, sm)

        def process(j, ph, wait_wb, start_next):
            m, sm = ms[ph]
            pltpu.make_async_copy(msg_hbm.at[pl.ds(base, CH)], m, sm).wait()
            pltpu.sync_copy(m, acc.at[idx_v.at[j]], add=True)
            if start_next:
                start(j + 2, ph)

        _pipeline(nch, start, process)
        plsc.subcore_barrier()
        pltpu.sync_copy(acc.at[pl.ds(row0, ROWS_PER_TILE)],
                        out_hbm.at[c, pl.ds(row0, ROWS_PER_TILE)])

    return scatter


_gather_h = _make_gather(EH_PAD)
_scatter_h = _make_scatter(EH_PAD)


BE = 2048


def _mlp_body(g_ref, ef_ref, mw1_ref, mb1_ref, mw2_ref, mb2_ref,
              aw1_ref, ab1_ref, aw2_ref, ab2_ref, out_ref):
    g = g_ref[...]
    ef = ef_ref[...]
    mw1 = mw1_ref[...]
    aw1 = aw1_ref[...]
    f32 = jnp.float32
    h = jnp.dot(g, mw1[:D], preferred_element_type=f32)
    h += jnp.dot(ef, mw1[D:], preferred_element_type=f32)
    h = jnp.maximum(h + mb1_ref[...], 0.0)
    m = jnp.dot(h, mw2_ref[...], preferred_element_type=f32) + mb2_ref[...]
    a = jnp.dot(g, aw1[:D], preferred_element_type=f32)
    a += jnp.dot(ef, aw1[D:], preferred_element_type=f32)
    a = jnp.maximum(a + ab1_ref[...], 0.0)
    w = jax.nn.sigmoid(
        jnp.dot(a, aw2_ref[...], preferred_element_type=f32) + ab2_ref[...])
    out_ref[...] = m * w


_DIN = D + DE
_full = lambda shape: pl.BlockSpec(shape, lambda i: (0,) * len(shape))

_mlp_call = pl.pallas_call(
    _mlp_body,
    grid=(EH_PAD // BE,),
    in_specs=[
        pl.BlockSpec((BE, D), lambda i: (i, 0)),
        pl.BlockSpec((BE, DE), lambda i: (i, 0)),
        _full((_DIN, D)),
        _full((1, D)),
        _full((D, D)),
        _full((1, D)),
        _full((_DIN, D)),
        _full((1, D)),
        _full((D, D)),
        _full((1, D)),
    ],
    out_specs=pl.BlockSpec((BE, D), lambda i: (i, 0)),
    out_shape=jax.ShapeDtypeStruct((EH_PAD, D), jnp.float32),
)


BN = 2000


def _gru_body(pa_ref, pb_ref, st_ref, wih_ref, whh_ref, bih_ref, bhh_ref,
              out_ref, outx_ref):
    f32 = jnp.float32
    sm = (pa_ref[0] + pa_ref[1]) + (pb_ref[0] + pb_ref[1])
    st = st_ref[...]
    gi = jnp.dot(sm, wih_ref[...], preferred_element_type=f32) + bih_ref[...]
    gh = jnp.dot(st, whh_ref[...], preferred_element_type=f32) + bhh_ref[...]
    r = jax.nn.sigmoid(gi[:, :D] + gh[:, :D])
    z = jax.nn.sigmoid(gi[:, D:2 * D] + gh[:, D:2 * D])
    n = jnp.tanh(gi[:, 2 * D:] + r * gh[:, 2 * D:])
    o = (1.0 - z) * n + z * st
    out_ref[...] = o
    outx_ref[...] = jnp.maximum(o, 0.0)


_gru_call = pl.pallas_call(
    _gru_body,
    grid=(N // BN,),
    in_specs=[
        pl.BlockSpec((NC, BN, D), lambda i: (0, i, 0)),
        pl.BlockSpec((NC, BN, D), lambda i: (0, i, 0)),
        pl.BlockSpec((BN, D), lambda i: (i, 0)),
        _full((D, 3 * D)),
        _full((D, 3 * D)),
        _full((1, 3 * D)),
        _full((1, 3 * D)),
    ],
    out_specs=[
        pl.BlockSpec((BN, D), lambda i: (i, 0)),
        pl.BlockSpec((BN, D), lambda i: (i, 0)),
    ],
    out_shape=[
        jax.ShapeDtypeStruct((N, D), jnp.float32),
        jax.ShapeDtypeStruct((N, D), jnp.float32),
    ],
)


def kernel(node_feat, edge, edge_feat,
           msg_W1, msg_b1, msg_W2, msg_b2,
           att_W1, att_b1, att_W2, att_b2,
           gru_Wih, gru_Whh, gru_bih, gru_bhh):
    per_w = EH_PAD // NW
    nch = per_w // CH
    pad = EH_PAD - EH
    ipad = jnp.zeros((pad,), jnp.int32)
    src = edge[:, 0]
    dst = edge[:, 1]
    srcA = jnp.concatenate([src[:EH], ipad]).reshape(NW, nch, CH)
    dstA = jnp.concatenate([dst[:EH], ipad]).reshape(NW, nch, CH)
    srcB = jnp.concatenate([src[EH:], ipad]).reshape(NW, nch, CH)
    dstB = jnp.concatenate([dst[EH:], ipad]).reshape(NW, nch, CH)
    fpad = jnp.zeros((pad, DE), jnp.float32)
    efA = jnp.concatenate([edge_feat[:EH], fpad])
    efB = jnp.concatenate([edge_feat[EH:], fpad])
    zeros = jnp.zeros((N_PAD, D), jnp.float32)
    x = node_feat
    out = node_feat
    for l in range(L):
        w = (msg_W1[l], msg_b1[l][None], msg_W2[l], msg_b2[l][None],
             att_W1[l], att_b1[l][None], att_W2[l], att_b2[l][None])
        gA = _gather_h(x, srcA, dstA)
        gB = _gather_h(x, srcB, dstB)
        msgA = _mlp_call(gA, efA, *w)
        msgB = _mlp_call(gB, efB, *w)
        pA = _scatter_h(msgA, dstA, zeros)
        pB = _scatter_h(msgB, dstB, zeros)
        out, x = _gru_call(pA, pB, x,
                           gru_Wih[l], gru_Whh[l],
                           gru_bih[l][None], gru_bhh[l][None])
    return out


# speedup vs baseline: 1.0468x; 1.0468x over previous
import functools

import jax
import jax.numpy as jnp
from jax import lax
from jax.experimental import pallas as pl
from jax.experimental.pallas import tpu as pltpu
from jax.experimental.pallas import tpu_sc as plsc

N = 10000
E = 320000
D = 128
DE = 16
L = 7

NC = 2
NS = 16
NW = NC * NS
EH = E // 2
CH = 40
EH_PAD = NW * 128 * CH
N_PAD = 10240
ROWS_PER_TILE = N_PAD // NS

_MESH = plsc.VectorSubcoreMesh(
    core_axis_name="c", subcore_axis_name="s", num_cores=NC, num_subcores=NS)


def _pipeline(nch, start, process):
    start(0, 0)
    start(1, 1)
    process(0, 0, False, True)
    process(1, 1, False, True)
    p_max = (nch - 4) // 2

    def body(p, carry):
        j = 2 * p
        process(j, 0, True, True)
        process(j + 1, 1, True, True)
        return carry

    lax.fori_loop(1, p_max + 1, body, 0)
    for j in range(2 * (p_max + 1), nch):
        process(j, j % 2, True, j + 2 < nch)


def _make_gather(e_cnt):
    per_w = e_cnt // NW
    nch = per_w // CH

    @functools.partial(
        pl.kernel,
        out_type=jax.ShapeDtypeStruct((e_cnt, D), jnp.float32),
        mesh=_MESH,
        scratch_types=[
            pltpu.VMEM((nch, CH), jnp.int32),
            pltpu.VMEM((nch, CH), jnp.int32),
            pltpu.VMEM((CH, D), jnp.float32),
            pltpu.VMEM((CH, D), jnp.float32),
            pltpu.VMEM((CH, D), jnp.float32),
            pltpu.VMEM((CH, D), jnp.float32),
            pltpu.VMEM((CH, D), jnp.float32),
            pltpu.VMEM((CH, D), jnp.float32),
            pltpu.SemaphoreType.DMA,
            pltpu.SemaphoreType.DMA,
            pltpu.SemaphoreType.DMA,
            pltpu.SemaphoreType.DMA,
            pltpu.SemaphoreType.DMA,
            pltpu.SemaphoreType.DMA,
        ],
    )
    def gather(x_hbm, src_hbm, dst_hbm, out_hbm,
               sidx, didx, a0, a1, b0, b1, o0, o1,
               sa0, sa1, sb0, sb1, so0, so1):
        c = lax.axis_index("c")
        s = lax.axis_index("s")
        wid = s * NC + c
        base = wid * per_w
        pltpu.sync_copy(src_hbm.at[wid, :, :], sidx)
        pltpu.sync_copy(dst_hbm.at[wid, :, :], didx)

        ab = ((a0, b0, o0, sa0, sb0, so0), (a1, b1, o1, sa1, sb1, so1))

        def start(j, ph):
            a, b, _, sa, sb, _ = ab[ph]
            pltpu.async_copy(x_hbm.at[sidx.at[j]], a, sa)
            pltpu.async_copy(x_hbm.at[didx.at[j]], b, sb)

        def process(j, ph, wait_wb, start_next):
            a, b, o, sa, sb, so = ab[ph]
            pltpu.make_async_copy(x_hbm.at[sidx.at[j]], a, sa).wait()
            pltpu.make_async_copy(x_hbm.at[didx.at[j]], b, sb).wait()
            if wait_wb:
                pltpu.make_async_copy(o, out_hbm.at[pl.ds(base, CH)],
                                      so).wait()

            def row(i, carry):
                for k in range(D // 16):
                    sl = pl.ds(k * 16, 16)
                    o[i, sl] = a[i, sl] - b[i, sl]
                return carry

            lax.fori_loop(0, CH, row, 0)
            if start_next:
                start(j + 2, ph)
            pltpu.async_copy(o, out_hbm.at[pl.ds(base + j * CH, CH)], so)

        _pipeline(nch, start, process)
        pltpu.make_async_copy(o0, out_hbm.at[pl.ds(base, CH)], so0).wait()
        pltpu.make_async_copy(o1, out_hbm.at[pl.ds(base, CH)], so1).wait()

    return gather


def _make_scatter(e_cnt):
    per_w = e_cnt // NW
    nch = per_w // CH

    @functools.partial(
        pl.kernel,
        out_type=jax.ShapeDtypeStruct((NC, N_PAD, D), jnp.float32),
        mesh=_MESH,
        scratch_types=[
            pltpu.VMEM((nch, CH), jnp.int32),
            pltpu.VMEM((CH, D), jnp.float32),
            pltpu.VMEM((CH, D), jnp.float32),
            pltpu.VMEM_SHARED((N_PAD, D), jnp.float32),
            pltpu.SemaphoreType.DMA,
            pltpu.SemaphoreType.DMA,
        ],
    )
    def scatter(msg_hbm, dst_hbm, zeros_hbm, out_hbm,
                idx_v, m0, m1, acc, sm0, sm1):
        c = lax.axis_index("c")
        s = lax.axis_index("s")
        wid = c * NS + s
        row0 = s * ROWS_PER_TILE
        base = wid * per_w
        pltpu.sync_copy(dst_hbm.at[wid, :, :], idx_v)
        pltpu.sync_copy(zeros_hbm.at[pl.ds(row0, ROWS_PER_TILE)],
                        acc.at[pl.ds(row0, ROWS_PER_TILE)])
        plsc.subcore_barrier()

        ms = ((m0, sm0), (m1, sm1))

        def start(j, ph):
            m, sm = ms[ph]
            pltpu.async_copy(msg_hbm.at[pl.ds(base + j * CH, CH)], m, sm)

        def process(j, ph, wait_wb, start_next):
            m, sm = ms[ph]
            pltpu.make_async_copy(msg_hbm.at[pl.ds(base, CH)], m, sm).wait()
            pltpu.sync_copy(m, acc.at[idx_v.at[j]], add=True)
            if start_next:
                start(j + 2, ph)

        _pipeline(nch, start, process)
        plsc.subcore_barrier()
        pltpu.sync_copy(acc.at[pl.ds(row0, ROWS_PER_TILE)],
                        out_hbm.at[c, pl.ds(row0, ROWS_PER_TILE)])

    return scatter


_gather_h = _make_gather(EH_PAD)
_scatter_h = _make_scatter(EH_PAD)


BE = 2048


def _mlp_body(g_ref, ef_ref, mw1_ref, mb1_ref, mw2_ref, mb2_ref,
              aw1_ref, ab1_ref, aw2_ref, ab2_ref, out_ref):
    g = g_ref[...]
    ef = ef_ref[...]
    mw1 = mw1_ref[...]
    aw1 = aw1_ref[...]
    f32 = jnp.float32
    h = jnp.dot(g, mw1[:D], preferred_element_type=f32)
    h += jnp.dot(ef, mw1[D:], preferred_element_type=f32)
    h = jnp.maximum(h + mb1_ref[...], 0.0)
    m = jnp.dot(h, mw2_ref[...], preferred_element_type=f32) + mb2_ref[...]
    a = jnp.dot(g, aw1[:D], preferred_element_type=f32)
    a += jnp.dot(ef, aw1[D:], preferred_element_type=f32)
    a = jnp.maximum(a + ab1_ref[...], 0.0)
    w = jax.nn.sigmoid(
        jnp.dot(a, aw2_ref[...], preferred_element_type=f32) + ab2_ref[...])
    out_ref[...] = m * w


_DIN = D + DE
_full = lambda shape: pl.BlockSpec(shape, lambda i: (0,) * len(shape))

_mlp_call = pl.pallas_call(
    _mlp_body,
    grid=(EH_PAD // BE,),
    in_specs=[
        pl.BlockSpec((BE, D), lambda i: (i, 0)),
        pl.BlockSpec((BE, DE), lambda i: (i, 0)),
        _full((_DIN, D)),
        _full((1, D)),
        _full((D, D)),
        _full((1, D)),
        _full((_DIN, D)),
        _full((1, D)),
        _full((D, D)),
        _full((1, D)),
    ],
    out_specs=pl.BlockSpec((BE, D), lambda i: (i, 0)),
    out_shape=jax.ShapeDtypeStruct((EH_PAD, D), jnp.float32),
)


BN = 2000


def _gru_body(pa_ref, pb_ref, st_ref, wih_ref, whh_ref, bih_ref, bhh_ref,
              out_ref, outx_ref):
    f32 = jnp.float32
    sm = (pa_ref[0] + pa_ref[1]) + (pb_ref[0] + pb_ref[1])
    st = st_ref[...]
    gi = jnp.dot(sm, wih_ref[...], preferred_element_type=f32) + bih_ref[...]
    gh = jnp.dot(st, whh_ref[...], preferred_element_type=f32) + bhh_ref[...]
    r = jax.nn.sigmoid(gi[:, :D] + gh[:, :D])
    z = jax.nn.sigmoid(gi[:, D:2 * D] + gh[:, D:2 * D])
    n = jnp.tanh(gi[:, 2 * D:] + r * gh[:, 2 * D:])
    o = (1.0 - z) * n + z * st
    out_ref[...] = o
    outx_ref[...] = jnp.maximum(o, 0.0)


_gru_call = pl.pallas_call(
    _gru_body,
    grid=(N // BN,),
    in_specs=[
        pl.BlockSpec((NC, BN, D), lambda i: (0, i, 0)),
        pl.BlockSpec((NC, BN, D), lambda i: (0, i, 0)),
        pl.BlockSpec((BN, D), lambda i: (i, 0)),
        _full((D, 3 * D)),
        _full((D, 3 * D)),
        _full((1, 3 * D)),
        _full((1, 3 * D)),
    ],
    out_specs=[
        pl.BlockSpec((BN, D), lambda i: (i, 0)),
        pl.BlockSpec((BN, D), lambda i: (i, 0)),
    ],
    out_shape=[
        jax.ShapeDtypeStruct((N, D), jnp.float32),
        jax.ShapeDtypeStruct((N, D), jnp.float32),
    ],
)


def kernel(node_feat, edge, edge_feat,
           msg_W1, msg_b1, msg_W2, msg_b2,
           att_W1, att_b1, att_W2, att_b2,
           gru_Wih, gru_Whh, gru_bih, gru_bhh):
    per_w = EH_PAD // NW
    nch = per_w // CH
    pad = EH_PAD - EH
    ipad = jnp.zeros((pad,), jnp.int32)
    src = edge[:, 0]
    dst = edge[:, 1]
    srcA = jnp.concatenate([src[:EH], ipad]).reshape(NW, nch, CH)
    dstA = jnp.concatenate([dst[:EH], ipad]).reshape(NW, nch, CH)
    srcB = jnp.concatenate([src[EH:], ipad]).reshape(NW, nch, CH)
    dstB = jnp.concatenate([dst[EH:], ipad]).reshape(NW, nch, CH)
    fpad = jnp.zeros((pad, DE), jnp.float32)
    efA = jnp.concatenate([edge_feat[:EH], fpad])
    efB = jnp.concatenate([edge_feat[EH:], fpad])
    zeros = jnp.zeros((N_PAD, D), jnp.float32)
    x = node_feat
    out = node_feat
    for l in range(L):
        w = (msg_W1[l], msg_b1[l][None], msg_W2[l], msg_b2[l][None],
             att_W1[l], att_b1[l][None], att_W2[l], att_b2[l][None])
        gA = _gather_h(x, srcA, dstA)
        gB = _gather_h(x, srcB, dstB)
        msgA = _mlp_call(gA, efA, *w)
        msgB = _mlp_call(gB, efB, *w)
        pA = _scatter_h(msgA, dstA, zeros)
        pB = _scatter_h(msgB, dstB, zeros)
        out, x = _gru_call(pA, pB, x,
                           gru_Wih[l], gru_Whh[l],
                           gru_bih[l][None], gru_bhh[l][None])
    return out


# speedup vs baseline: 2.5115x; 2.3991x over previous
import functools

import jax
import jax.numpy as jnp
from jax import lax
from jax.experimental import pallas as pl
from jax.experimental.pallas import tpu as pltpu
from jax.experimental.pallas import tpu_sc as plsc

N = 10000
E = 320000
D = 128
DE = 16
L = 7

NC = 2
NS = 16
NW = NC * NS
EH = E // 2
CH = 40
EH_PAD = NW * 128 * CH
N_PAD = 10240
ROWS_PER_TILE = N_PAD // NS

_MESH = plsc.VectorSubcoreMesh(
    core_axis_name="c", subcore_axis_name="s", num_cores=NC, num_subcores=NS)


def _pipeline(nch, start, process):
    start(0, 0)
    start(1, 1)
    process(0, 0, False, True)
    process(1, 1, False, True)
    p_max = (nch - 4) // 2

    def body(p, carry):
        j = 2 * p
        process(j, 0, True, True)
        process(j + 1, 1, True, True)
        return carry

    lax.fori_loop(1, p_max + 1, body, 0)
    for j in range(2 * (p_max + 1), nch):
        process(j, j % 2, True, j + 2 < nch)


def _make_gather(e_cnt):
    per_w = e_cnt // NW
    nch = per_w // CH

    @functools.partial(
        pl.kernel,
        out_type=jax.ShapeDtypeStruct((e_cnt, D), jnp.float32),
        mesh=_MESH,
        scratch_types=[
            pltpu.VMEM((nch, CH), jnp.int32),
            pltpu.VMEM((nch, CH), jnp.int32),
            pltpu.VMEM((CH, D), jnp.float32),
            pltpu.VMEM((CH, D), jnp.float32),
            pltpu.VMEM((CH, D), jnp.float32),
            pltpu.VMEM((CH, D), jnp.float32),
            pltpu.VMEM((CH, D), jnp.float32),
            pltpu.VMEM((CH, D), jnp.float32),
            pltpu.SemaphoreType.DMA,
            pltpu.SemaphoreType.DMA,
            pltpu.SemaphoreType.DMA,
            pltpu.SemaphoreType.DMA,
            pltpu.SemaphoreType.DMA,
            pltpu.SemaphoreType.DMA,
        ],
    )
    def gather(x_hbm, src_hbm, dst_hbm, out_hbm,
               sidx, didx, a0, a1, b0, b1, o0, o1,
               sa0, sa1, sb0, sb1, so0, so1):
        c = lax.axis_index("c")
        s = lax.axis_index("s")
        wid = s * NC + c
        base = wid * per_w
        pltpu.sync_copy(src_hbm.at[wid, :, :], sidx)
        pltpu.sync_copy(dst_hbm.at[wid, :, :], didx)

        ab = ((a0, b0, o0, sa0, sb0, so0), (a1, b1, o1, sa1, sb1, so1))

        def start(j, ph):
            a, b, _, sa, sb, _ = ab[ph]
            pltpu.async_copy(x_hbm.at[sidx.at[j]], a, sa)
            pltpu.async_copy(x_hbm.at[didx.at[j]], b, sb)

        def process(j, ph, wait_wb, start_next):
            a, b, o, sa, sb, so = ab[ph]
            pltpu.make_async_copy(x_hbm.at[sidx.at[j]], a, sa).wait()
            pltpu.make_async_copy(x_hbm.at[didx.at[j]], b, sb).wait()
            if wait_wb:
                pltpu.make_async_copy(o, out_hbm.at[pl.ds(base, CH)],
                                      so).wait()

            def row(i, carry):
                for k in range(D // 16):
                    sl = pl.ds(k * 16, 16)
                    o[i, sl] = a[i, sl] - b[i, sl]
                return carry

            lax.fori_loop(0, CH, row, 0)
            if start_next:
                start(j + 2, ph)
            pltpu.async_copy(o, out_hbm.at[pl.ds(base + j * CH, CH)], so)

        _pipeline(nch, start, process)
        pltpu.make_async_copy(o0, out_hbm.at[pl.ds(base, CH)], so0).wait()
        pltpu.make_async_copy(o1, out_hbm.at[pl.ds(base, CH)], so1).wait()

    return gather


def _make_scatter(e_cnt):
    per_w = e_cnt // NW
    nch = per_w // CH

    @functools.partial(
        pl.kernel,
        out_type=jax.ShapeDtypeStruct((NC, N_PAD, D), jnp.float32),
        mesh=_MESH,
        scratch_types=[
            pltpu.VMEM((nch, CH), jnp.int32),
            pltpu.VMEM((CH, D), jnp.float32),
            pltpu.VMEM((CH, D), jnp.float32),
            pltpu.VMEM_SHARED((N_PAD, D), jnp.float32),
            pltpu.SemaphoreType.DMA,
            pltpu.SemaphoreType.DMA,
        ],
    )
    def scatter(msg_hbm, dst_hbm, zeros_hbm, out_hbm,
                idx_v, m0, m1, acc, sm0, sm1):
        c = lax.axis_index("c")
        s = lax.axis_index("s")
        wid = c * NS + s
        row0 = s * ROWS_PER_TILE
        base = wid * per_w
        pltpu.sync_copy(dst_hbm.at[wid, :, :], idx_v)
        pltpu.sync_copy(zeros_hbm.at[pl.ds(row0, ROWS_PER_TILE)],
                        acc.at[pl.ds(row0, ROWS_PER_TILE)])
        plsc.subcore_barrier()

        ms = ((m0, sm0), (m1, sm1))

        def start(j, ph):
            m, sm = ms[ph]
            pltpu.async_copy(msg_hbm.at[pl.ds(base + j * CH, CH)], m, sm)

        def process(j, ph, wait_wb, start_next):
            m, sm = ms[ph]
            pltpu.make_async_copy(msg_hbm.at[pl.ds(base, CH)], m, sm).wait()
            pltpu.sync_copy(m, acc.at[idx_v.at[j]], add=True)
            if start_next:
                start(j + 2, ph)

        _pipeline(nch, start, process)
        plsc.subcore_barrier()
        pltpu.sync_copy(acc.at[pl.ds(row0, ROWS_PER_TILE)],
                        out_hbm.at[c, pl.ds(row0, ROWS_PER_TILE)])

    return scatter


_gather_h = _make_gather(EH_PAD)
_scatter_h = _make_scatter(EH_PAD)


BE = 2048


def _mlp_body(g_ref, ef_ref, mw1_ref, mb1_ref, mw2_ref, mb2_ref,
              aw1_ref, ab1_ref, aw2_ref, ab2_ref, out_ref):
    g = g_ref[...]
    ef = ef_ref[...]
    mw1 = mw1_ref[...]
    aw1 = aw1_ref[...]
    f32 = jnp.float32
    h = jnp.dot(g, mw1[:D], preferred_element_type=f32)
    h += jnp.dot(ef, mw1[D:], preferred_element_type=f32)
    h = jnp.maximum(h + mb1_ref[...], 0.0)
    m = jnp.dot(h, mw2_ref[...], preferred_element_type=f32) + mb2_ref[...]
    a = jnp.dot(g, aw1[:D], preferred_element_type=f32)
    a += jnp.dot(ef, aw1[D:], preferred_element_type=f32)
    a = jnp.maximum(a + ab1_ref[...], 0.0)
    w = jax.nn.sigmoid(
        jnp.dot(a, aw2_ref[...], preferred_element_type=f32) + ab2_ref[...])
    out_ref[...] = m * w


_DIN = D + DE
_full = lambda shape: pl.BlockSpec(shape, lambda i: (0,) * len(shape))

_mlp_call = pl.pallas_call(
    _mlp_body,
    grid=(EH_PAD // BE,),
    in_specs=[
        pl.BlockSpec((BE, D), lambda i: (i, 0)),
        pl.BlockSpec((BE, DE), lambda i: (i, 0)),
        _full((_DIN, D)),
        _full((1, D)),
        _full((D, D)),
        _full((1, D)),
        _full((_DIN, D)),
        _full((1, D)),
        _full((D, D)),
        _full((1, D)),
    ],
    out_specs=pl.BlockSpec((BE, D), lambda i: (i, 0)),
    out_shape=jax.ShapeDtypeStruct((EH_PAD, D), jnp.float32),
)


BN = 2000


def _gru_body(pa_ref, pb_ref, st_ref, wih_ref, whh_ref, bih_ref, bhh_ref,
              out_ref, outx_ref):
    f32 = jnp.float32
    sm = (pa_ref[0] + pa_ref[1]) + (pb_ref[0] + pb_ref[1])
    st = st_ref[...]
    gi = jnp.dot(sm, wih_ref[...], preferred_element_type=f32) + bih_ref[...]
    gh = jnp.dot(st, whh_ref[...], preferred_element_type=f32) + bhh_ref[...]
    r = jax.nn.sigmoid(gi[:, :D] + gh[:, :D])
    z = jax.nn.sigmoid(gi[:, D:2 * D] + gh[:, D:2 * D])
    n = jnp.tanh(gi[:, 2 * D:] + r * gh[:, 2 * D:])
    o = (1.0 - z) * n + z * st
    out_ref[...] = o
    outx_ref[...] = jnp.maximum(o, 0.0)


_gru_call = pl.pallas_call(
    _gru_body,
    grid=(N // BN,),
    in_specs=[
        pl.BlockSpec((NC, BN, D), lambda i: (0, i, 0)),
        pl.BlockSpec((NC, BN, D), lambda i: (0, i, 0)),
        pl.BlockSpec((BN, D), lambda i: (i, 0)),
        _full((D, 3 * D)),
        _full((D, 3 * D)),
        _full((1, 3 * D)),
        _full((1, 3 * D)),
    ],
    out_specs=[
        pl.BlockSpec((BN, D), lambda i: (i, 0)),
        pl.BlockSpec((BN, D), lambda i: (i, 0)),
    ],
    out_shape=[
        jax.ShapeDtypeStruct((N, D), jnp.float32),
        jax.ShapeDtypeStruct((N, D), jnp.float32),
    ],
)


def kernel(node_feat, edge, edge_feat,
           msg_W1, msg_b1, msg_W2, msg_b2,
           att_W1, att_b1, att_W2, att_b2,
           gru_Wih, gru_Whh, gru_bih, gru_bhh):
    per_w = EH_PAD // NW
    nch = per_w // CH
    pad = EH_PAD - EH
    ipad = jnp.arange(pad, dtype=jnp.int32) % N
    src = edge[:, 0]
    dst = edge[:, 1]
    srcA = jnp.concatenate([src[:EH], ipad]).reshape(NW, nch, CH)
    dstA = jnp.concatenate([dst[:EH], ipad]).reshape(NW, nch, CH)
    srcB = jnp.concatenate([src[EH:], ipad]).reshape(NW, nch, CH)
    dstB = jnp.concatenate([dst[EH:], ipad]).reshape(NW, nch, CH)
    fpad = jnp.zeros((pad, DE), jnp.float32)
    efA = jnp.concatenate([edge_feat[:EH], fpad])
    efB = jnp.concatenate([edge_feat[EH:], fpad])
    zeros = jnp.zeros((N_PAD, D), jnp.float32)
    x = node_feat
    out = node_feat
    for l in range(L):
        w = (msg_W1[l], msg_b1[l][None], msg_W2[l], msg_b2[l][None],
             att_W1[l], att_b1[l][None], att_W2[l], att_b2[l][None])
        gA = _gather_h(x, srcA, dstA)
        gB = _gather_h(x, srcB, dstB)
        msgA = _mlp_call(gA, efA, *w)
        msgB = _mlp_call(gB, efB, *w)
        pA = _scatter_h(msgA, dstA, zeros)
        pB = _scatter_h(msgB, dstB, zeros)
        out, x = _gru_call(pA, pB, x,
                           gru_Wih[l], gru_Whh[l],
                           gru_bih[l][None], gru_bhh[l][None])
    return out


# speedup vs baseline: 2.5869x; 1.0300x over previous
import functools

import jax
import jax.numpy as jnp
from jax import lax
from jax.experimental import pallas as pl
from jax.experimental.pallas import tpu as pltpu
from jax.experimental.pallas import tpu_sc as plsc

N = 10000
E = 320000
D = 128
DE = 16
L = 7

NC = 2
NS = 16
NW = NC * NS
EH = E // 2
CH = 40
EH_PAD = EH
N_PAD = 10240
ROWS_PER_TILE = N_PAD // NS

_MESH = plsc.VectorSubcoreMesh(
    core_axis_name="c", subcore_axis_name="s", num_cores=NC, num_subcores=NS)


def _pipeline(nch, start, process):
    start(0, 0)
    start(1, 1)
    process(0, 0, False, True)
    process(1, 1, False, True)
    p_max = (nch - 4) // 2

    def body(p, carry):
        j = 2 * p
        process(j, 0, True, True)
        process(j + 1, 1, True, True)
        return carry

    lax.fori_loop(1, p_max + 1, body, 0)
    for j in range(2 * (p_max + 1), nch):
        process(j, j % 2, True, j + 2 < nch)


def _make_gather(e_cnt):
    per_w = e_cnt // NW
    nch = per_w // CH

    @functools.partial(
        pl.kernel,
        out_type=jax.ShapeDtypeStruct((e_cnt, D), jnp.float32),
        mesh=_MESH,
        scratch_types=[
            pltpu.VMEM((nch, CH), jnp.int32),
            pltpu.VMEM((nch, CH), jnp.int32),
            pltpu.VMEM((CH, D), jnp.float32),
            pltpu.VMEM((CH, D), jnp.float32),
            pltpu.VMEM((CH, D), jnp.float32),
            pltpu.VMEM((CH, D), jnp.float32),
            pltpu.VMEM((CH, D), jnp.float32),
            pltpu.VMEM((CH, D), jnp.float32),
            pltpu.SemaphoreType.DMA,
            pltpu.SemaphoreType.DMA,
            pltpu.SemaphoreType.DMA,
            pltpu.SemaphoreType.DMA,
            pltpu.SemaphoreType.DMA,
            pltpu.SemaphoreType.DMA,
        ],
    )
    def gather(x_hbm, src_hbm, dst_hbm, out_hbm,
               sidx, didx, a0, a1, b0, b1, o0, o1,
               sa0, sa1, sb0, sb1, so0, so1):
        c = lax.axis_index("c")
        s = lax.axis_index("s")
        wid = s * NC + c
        base = wid * per_w
        pltpu.sync_copy(src_hbm.at[wid, :, :], sidx)
        pltpu.sync_copy(dst_hbm.at[wid, :, :], didx)

        ab = ((a0, b0, o0, sa0, sb0, so0), (a1, b1, o1, sa1, sb1, so1))

        def start(j, ph):
            a, b, _, sa, sb, _ = ab[ph]
            pltpu.async_copy(x_hbm.at[sidx.at[j]], a, sa)
            pltpu.async_copy(x_hbm.at[didx.at[j]], b, sb)

        def process(j, ph, wait_wb, start_next):
            a, b, o, sa, sb, so = ab[ph]
            pltpu.make_async_copy(x_hbm.at[sidx.at[j]], a, sa).wait()
            pltpu.make_async_copy(x_hbm.at[didx.at[j]], b, sb).wait()
            if wait_wb:
                pltpu.make_async_copy(o, out_hbm.at[pl.ds(base, CH)],
                                      so).wait()

            def row(i, carry):
                for k in range(D // 16):
                    sl = pl.ds(k * 16, 16)
                    o[i, sl] = a[i, sl] - b[i, sl]
                return carry

            lax.fori_loop(0, CH, row, 0)
            if start_next:
                start(j + 2, ph)
            pltpu.async_copy(o, out_hbm.at[pl.ds(base + j * CH, CH)], so)

        _pipeline(nch, start, process)
        pltpu.make_async_copy(o0, out_hbm.at[pl.ds(base, CH)], so0).wait()
        pltpu.make_async_copy(o1, out_hbm.at[pl.ds(base, CH)], so1).wait()

    return gather


def _make_scatter(e_cnt):
    per_w = e_cnt // NW
    nch = per_w // CH

    @functools.partial(
        pl.kernel,
        out_type=jax.ShapeDtypeStruct((NC, N_PAD, D), jnp.float32),
        mesh=_MESH,
        scratch_types=[
            pltpu.VMEM((nch, CH), jnp.int32),
            pltpu.VMEM((CH, D), jnp.float32),
            pltpu.VMEM((CH, D), jnp.float32),
            pltpu.VMEM_SHARED((N_PAD, D), jnp.float32),
            pltpu.SemaphoreType.DMA,
            pltpu.SemaphoreType.DMA,
        ],
    )
    def scatter(msg_hbm, dst_hbm, zeros_hbm, out_hbm,
                idx_v, m0, m1, acc, sm0, sm1):
        c = lax.axis_index("c")
        s = lax.axis_index("s")
        wid = c * NS + s
        row0 = s * ROWS_PER_TILE
        base = wid * per_w
        pltpu.sync_copy(dst_hbm.at[wid, :, :], idx_v)
        pltpu.sync_copy(zeros_hbm.at[pl.ds(row0, ROWS_PER_TILE)],
                        acc.at[pl.ds(row0, ROWS_PER_TILE)])
        plsc.subcore_barrier()

        ms = ((m0, sm0), (m1, sm1))

        def start(j, ph):
            m, sm = ms[ph]
            pltpu.async_copy(msg_hbm.at[pl.ds(base + j * CH, CH)], m, sm)

        def process(j, ph, wait_wb, start_next):
            m, sm = ms[ph]
            pltpu.make_async_copy(msg_hbm.at[pl.ds(base, CH)], m, sm).wait()
            pltpu.sync_copy(m, acc.at[idx_v.at[j]], add=True)
            if start_next:
                start(j + 2, ph)

        _pipeline(nch, start, process)
        plsc.subcore_barrier()
        pltpu.sync_copy(acc.at[pl.ds(row0, ROWS_PER_TILE)],
                        out_hbm.at[c, pl.ds(row0, ROWS_PER_TILE)])

    return scatter


_gather_h = _make_gather(EH_PAD)
_scatter_h = _make_scatter(EH_PAD)


BE = 2000


def _mlp_body(g_ref, ef_ref, mw1_ref, mb1_ref, mw2_ref, mb2_ref,
              aw1_ref, ab1_ref, aw2_ref, ab2_ref, out_ref):
    g = g_ref[...]
    ef = ef_ref[...]
    mw1 = mw1_ref[...]
    aw1 = aw1_ref[...]
    f32 = jnp.float32
    h = jnp.dot(g, mw1[:D], preferred_element_type=f32)
    h += jnp.dot(ef, mw1[D:], preferred_element_type=f32)
    h = jnp.maximum(h + mb1_ref[...], 0.0)
    m = jnp.dot(h, mw2_ref[...], preferred_element_type=f32) + mb2_ref[...]
    a = jnp.dot(g, aw1[:D], preferred_element_type=f32)
    a += jnp.dot(ef, aw1[D:], preferred_element_type=f32)
    a = jnp.maximum(a + ab1_ref[...], 0.0)
    w = jax.nn.sigmoid(
        jnp.dot(a, aw2_ref[...], preferred_element_type=f32) + ab2_ref[...])
    out_ref[...] = m * w


_DIN = D + DE
_full = lambda shape: pl.BlockSpec(shape, lambda i: (0,) * len(shape))

_mlp_call = pl.pallas_call(
    _mlp_body,
    grid=(EH_PAD // BE,),
    in_specs=[
        pl.BlockSpec((BE, D), lambda i: (i, 0)),
        pl.BlockSpec((BE, DE), lambda i: (i, 0)),
        _full((_DIN, D)),
        _full((1, D)),
        _full((D, D)),
        _full((1, D)),
        _full((_DIN, D)),
        _full((1, D)),
        _full((D, D)),
        _full((1, D)),
    ],
    out_specs=pl.BlockSpec((BE, D), lambda i: (i, 0)),
    out_shape=jax.ShapeDtypeStruct((EH_PAD, D), jnp.float32),
)


BN = 2000


def _gru_body(pa_ref, pb_ref, st_ref, wih_ref, whh_ref, bih_ref, bhh_ref,
              out_ref, outx_ref):
    f32 = jnp.float32
    sm = (pa_ref[0] + pa_ref[1]) + (pb_ref[0] + pb_ref[1])
    st = st_ref[...]
    gi = jnp.dot(sm, wih_ref[...], preferred_element_type=f32) + bih_ref[...]
    gh = jnp.dot(st, whh_ref[...], preferred_element_type=f32) + bhh_ref[...]
    r = jax.nn.sigmoid(gi[:, :D] + gh[:, :D])
    z = jax.nn.sigmoid(gi[:, D:2 * D] + gh[:, D:2 * D])
    n = jnp.tanh(gi[:, 2 * D:] + r * gh[:, 2 * D:])
    o = (1.0 - z) * n + z * st
    out_ref[...] = o
    outx_ref[...] = jnp.maximum(o, 0.0)


_gru_call = pl.pallas_call(
    _gru_body,
    grid=(N // BN,),
    in_specs=[
        pl.BlockSpec((NC, BN, D), lambda i: (0, i, 0)),
        pl.BlockSpec((NC, BN, D), lambda i: (0, i, 0)),
        pl.BlockSpec((BN, D), lambda i: (i, 0)),
        _full((D, 3 * D)),
        _full((D, 3 * D)),
        _full((1, 3 * D)),
        _full((1, 3 * D)),
    ],
    out_specs=[
        pl.BlockSpec((BN, D), lambda i: (i, 0)),
        pl.BlockSpec((BN, D), lambda i: (i, 0)),
    ],
    out_shape=[
        jax.ShapeDtypeStruct((N, D), jnp.float32),
        jax.ShapeDtypeStruct((N, D), jnp.float32),
    ],
)


def kernel(node_feat, edge, edge_feat,
           msg_W1, msg_b1, msg_W2, msg_b2,
           att_W1, att_b1, att_W2, att_b2,
           gru_Wih, gru_Whh, gru_bih, gru_bhh):
    per_w = EH_PAD // NW
    nch = per_w // CH
    src = edge[:, 0]
    dst = edge[:, 1]
    srcA = src[:EH].reshape(NW, nch, CH)
    dstA = dst[:EH].reshape(NW, nch, CH)
    srcB = src[EH:].reshape(NW, nch, CH)
    dstB = dst[EH:].reshape(NW, nch, CH)
    efA = edge_feat[:EH]
    efB = edge_feat[EH:]
    zeros = jnp.zeros((N_PAD, D), jnp.float32)
    x = node_feat
    out = node_feat
    for l in range(L):
        w = (msg_W1[l], msg_b1[l][None], msg_W2[l], msg_b2[l][None],
             att_W1[l], att_b1[l][None], att_W2[l], att_b2[l][None])
        gA = _gather_h(x, srcA, dstA)
        gB = _gather_h(x, srcB, dstB)
        msgA = _mlp_call(gA, efA, *w)
        msgB = _mlp_call(gB, efB, *w)
        pA = _scatter_h(msgA, dstA, zeros)
        pB = _scatter_h(msgB, dstB, zeros)
        out, x = _gru_call(pA, pB, x,
                           gru_Wih[l], gru_Whh[l],
                           gru_bih[l][None], gru_bhh[l][None])
    return out
